# baseline (speedup 1.0000x reference)
import functools

import jax
import jax.numpy as jnp
from jax import lax
from jax.experimental import pallas as pl
from jax.experimental.pallas import tpu as pltpu
from jax.experimental.pallas import tpu_sc as plsc

_NUM_ATTRS = 26
_VOCAB = 100000
_EMBED_DIM = 32
_BATCH = 16384

_NW = 32
_ROWS_PER_W = _NUM_ATTRS * _EMBED_DIM // _NW
_Q = 4096
_NQ = _BATCH // _Q

_mesh = plsc.VectorSubcoreMesh(core_axis_name="c", subcore_axis_name="s")


@functools.partial(
    pl.kernel,
    out_type=jax.ShapeDtypeStruct((_NUM_ATTRS * _EMBED_DIM, _BATCH),
                                  jnp.float32),
    mesh=_mesh,
    scratch_types=[
        pltpu.VMEM((_VOCAB,), jnp.float32),
        pltpu.VMEM((_BATCH,), jnp.int32),
        pltpu.VMEM((2, _Q), jnp.float32),
        pltpu.SemaphoreType.DMA((2,)),
        pltpu.SemaphoreType.DMA,
    ],
    compiler_params=pltpu.CompilerParams(needs_layout_passes=False),
)
def _lookup(tup_hbm, tab_hbm, out_hbm, row_v, idx_v, outb_v, osem, rsem):
  w = lax.axis_index("s") * 2 + lax.axis_index("c")
  r0 = w * _ROWS_PER_W

  def per_row(i, prev_a):
    r = r0 + i
    a = r // _EMBED_DIM
    e = r % _EMBED_DIM

    pltpu.async_copy(tab_hbm.at[a, e], row_v, rsem)

    @pl.when(a != prev_a)
    def _():
      pltpu.sync_copy(tup_hbm.at[a], idx_v)

    pltpu.make_async_copy(tab_hbm.at[a, e], row_v, rsem).wait()

    for q in range(_NQ):
      slot = q % 2

      @pl.when(i * _NQ + q >= 2)
      def _():
        pltpu.make_async_copy(
            outb_v.at[slot], out_hbm.at[r, pl.ds(0, _Q)], osem.at[slot]
        ).wait()

      @plsc.parallel_loop(0, _Q // 16, 1, unroll=16)
      def _(k):
        iv = idx_v[pl.ds(q * _Q + k * 16, 16)]
        outb_v[slot, pl.ds(k * 16, 16)] = plsc.load_gather(row_v, [iv])

      pltpu.async_copy(
          outb_v.at[slot], out_hbm.at[r, pl.ds(q * _Q, _Q)], osem.at[slot])
    return a

  lax.fori_loop(0, _ROWS_PER_W, per_row, -1)
  for slot in range(2):
    pltpu.make_async_copy(
        outb_v.at[slot], out_hbm.at[0, pl.ds(0, _Q)], osem.at[slot]).wait()


def kernel(tuples, tables):
  tup_t = tuples.astype(jnp.int32).T
  tab_t = jnp.transpose(tables, (0, 2, 1))
  out_t = _lookup(tup_t, tab_t)
  return out_t.T


# speedup vs baseline: 1.0038x; 1.0038x over previous
import functools

import jax
import jax.numpy as jnp
from jax import lax
from jax.experimental import pallas as pl
from jax.experimental.pallas import tpu as pltpu
from jax.experimental.pallas import tpu_sc as plsc

_NUM_ATTRS = 26
_VOCAB = 100000
_EMBED_DIM = 32
_BATCH = 16384

_NW = 32
_ROWS_PER_W = _NUM_ATTRS * _EMBED_DIM // _NW
_Q = 4096
_NQ = _BATCH // _Q

_mesh = plsc.VectorSubcoreMesh(core_axis_name="c", subcore_axis_name="s")


@functools.partial(
    pl.kernel,
    out_type=jax.ShapeDtypeStruct((_NUM_ATTRS * _EMBED_DIM, _BATCH),
                                  jnp.float32),
    mesh=_mesh,
    scratch_types=[
        pltpu.VMEM((_VOCAB,), jnp.float32),
        pltpu.VMEM((_BATCH,), jnp.int32),
        pltpu.VMEM((2, _Q), jnp.float32),
        pltpu.SemaphoreType.DMA((2,)),
    ],
    compiler_params=pltpu.CompilerParams(needs_layout_passes=False),
)
def _lookup(tup_hbm, tab_hbm, out_hbm, row_v, idx_v, outb_v, osem):
  w = lax.axis_index("s") * 2 + lax.axis_index("c")
  r0 = w * _ROWS_PER_W

  def per_row(i, prev_a):
    r = r0 + i
    a = r // _EMBED_DIM
    e = r % _EMBED_DIM

    @pl.when(a != prev_a)
    def _():
      pltpu.sync_copy(tup_hbm.at[a], idx_v)

    pltpu.sync_copy(tab_hbm.at[a, e], row_v)

    for q in range(_NQ):
      slot = q % 2

      @pl.when(i * _NQ + q >= 2)
      def _():
        pltpu.make_async_copy(
            outb_v.at[slot], out_hbm.at[r, pl.ds(0, _Q)], osem.at[slot]
        ).wait()

      @plsc.parallel_loop(0, _Q // 16, 1, unroll=16)
      def _(k):
        iv = idx_v[pl.ds(q * _Q + k * 16, 16)]
        outb_v[slot, pl.ds(k * 16, 16)] = plsc.load_gather(row_v, [iv])

      pltpu.async_copy(
          outb_v.at[slot], out_hbm.at[r, pl.ds(q * _Q, _Q)], osem.at[slot])
    return a

  lax.fori_loop(0, _ROWS_PER_W, per_row, -1)
  for slot in range(2):
    pltpu.make_async_copy(
        outb_v.at[slot], out_hbm.at[0, pl.ds(0, _Q)], osem.at[slot]).wait()


def kernel(tuples, tables):
  tup_t = tuples.astype(jnp.int32).T
  tab_t = jnp.transpose(tables, (0, 2, 1))
  out_t = _lookup(tup_t, tab_t)
  return out_t.T
